# Initial kernel scaffold; baseline (speedup 1.0000x reference)
import jax
import jax.numpy as jnp
from jax.experimental import pallas as pl


def kernel(node_feat, edge_feat, edge_index, triplets, h_table, e_table, Wmsg, Wself, We, rel_bias, b_h, b_e):
    raise NotImplementedError("write your pallas kernel here")


# speedup vs baseline: 2.3258x; 2.3258x over previous
import functools

import jax
import jax.numpy as jnp
from jax import lax
from jax.experimental import pallas as pl
from jax.experimental.pallas import tpu as pltpu
from jax.experimental.pallas import tpu_sc as plsc

N = 10000
E = 320000
HID = 128
NTYPE = 16

NC = 2
NS = 16
NW = NC * NS
C = 128

Np = 10240
Ep = NW * C * 79
BE = 2048
BN = 2048

_MESH = dict(core_axis_name="c", subcore_axis_name="s")


def _sc_gather(table, idx, B, chunk):
    nb = B // NW
    nchunks = nb // chunk

    @functools.partial(
        pl.kernel,
        mesh=plsc.VectorSubcoreMesh(**_MESH),
        out_type=jax.ShapeDtypeStruct((B, HID), jnp.float32),
        scratch_types=[
            pltpu.VMEM((chunk,), jnp.int32),
            pltpu.VMEM((chunk, HID), jnp.float32),
            pltpu.SemaphoreType.DMA,
        ],
    )
    def gk(table_hbm, idx_hbm, out_hbm, idx_v, rows_v, sem):
        wid = lax.axis_index("c") * NS + lax.axis_index("s")
        base_w = wid * nb

        def body(i, carry):
            base = pl.multiple_of(base_w + i * chunk, 8)
            pltpu.sync_copy(idx_hbm.at[pl.ds(base, chunk)], idx_v)
            pltpu.async_copy(table_hbm.at[idx_v], rows_v, sem).wait()
            pltpu.sync_copy(rows_v, out_hbm.at[pl.ds(base, chunk)])
            return carry

        lax.fori_loop(0, nchunks, body, 0)

    return gk(table, idx)


def _sc_scatter_add(msg, dst, zeros_stripe):
    nb = Ep // NW
    nchunks = nb // C
    stripe = Np // NS

    @functools.partial(
        pl.kernel,
        mesh=plsc.VectorSubcoreMesh(**_MESH),
        out_type=jax.ShapeDtypeStruct((NC, Np, HID), jnp.float32),
        scratch_types=[
            pltpu.VMEM((C,), jnp.int32),
            pltpu.VMEM((C, HID), jnp.float32),
            pltpu.VMEM_SHARED((Np, HID), jnp.float32),
        ],
    )
    def sk(msg_hbm, dst_hbm, zeros_hbm, out_hbm, idx_v, rows_v, acc_sh):
        c = lax.axis_index("c")
        s = lax.axis_index("s")
        wid = c * NS + s

        pltpu.sync_copy(zeros_hbm, acc_sh.at[pl.ds(s * stripe, stripe)])
        plsc.subcore_barrier()

        base_w = wid * nb

        def body(i, carry):
            base = pl.multiple_of(base_w + i * C, 8)
            pltpu.sync_copy(dst_hbm.at[pl.ds(base, C)], idx_v)
            pltpu.sync_copy(msg_hbm.at[pl.ds(base, C)], rows_v)
            pltpu.sync_copy(rows_v, acc_sh.at[idx_v], add=True)
            return carry

        lax.fori_loop(0, nchunks, body, 0)
        plsc.subcore_barrier()
        pltpu.sync_copy(acc_sh.at[pl.ds(s * stripe, stripe)],
                        out_hbm.at[c, pl.ds(s * stripe, stripe)])

    return sk(msg, dst, zeros_stripe)


def _dot(a, b):
    return jnp.dot(a, b, preferred_element_type=jnp.float32)


def _one_hot(et2):
    t = lax.broadcasted_iota(jnp.int32, (1, NTYPE), 1)
    return (et2 == t).astype(jnp.float32)


def _msg0_body(hs_ref, et_ref, etab_ref, rb_ref, w_ref, out_ref):
    pid = pl.program_id(0)
    oh = _one_hot(et_ref[0])
    e0 = _dot(oh, etab_ref[...])
    rb = _dot(oh, rb_ref[...])
    x = hs_ref[...] * e0 + rb
    msg = _dot(x, w_ref[...])
    rowid = pid * BE + lax.broadcasted_iota(jnp.int32, (BE, 1), 0)
    out_ref[...] = jnp.where(rowid < E, msg, 0.0)


def _msg0(hs0, et3, e_tab, rb0, w0):
    nblk = Ep // BE
    return pl.pallas_call(
        _msg0_body,
        grid=(nblk,),
        in_specs=[
            pl.BlockSpec((BE, HID), lambda i: (i, 0)),
            pl.BlockSpec((1, BE, 1), lambda i: (i, 0, 0)),
            pl.BlockSpec((NTYPE, HID), lambda i: (0, 0)),
            pl.BlockSpec((NTYPE, HID), lambda i: (0, 0)),
            pl.BlockSpec((HID, HID), lambda i: (0, 0)),
        ],
        out_specs=pl.BlockSpec((BE, HID), lambda i: (i, 0)),
        out_shape=jax.ShapeDtypeStruct((Ep, HID), jnp.float32),
    )(hs0, et3, e_tab, rb0, w0)


def _msg1_body(hs0_ref, hs1_ref, hd1_ref, et_ref, etab_ref, rb0_ref, we_ref,
               be_ref, rb1_ref, w1_ref, out_ref):
    pid = pl.program_id(0)
    oh = _one_hot(et_ref[0])
    e0 = _dot(oh, etab_ref[...])
    t = _dot(e0 + hs0_ref[...] + hd1_ref[...], we_ref[...]) + be_ref[...]
    e1 = e0 + jnp.maximum(t, 0.0)
    x = hs1_ref[...] * e1 + _dot(oh, rb1_ref[...])
    msg = _dot(x, w1_ref[...])
    rowid = pid * BE + lax.broadcasted_iota(jnp.int32, (BE, 1), 0)
    out_ref[...] = jnp.where(rowid < E, msg, 0.0)


def _msg1(hs0, hs1, hd1, et3, e_tab, rb0, we0, be0, rb1, w1):
    nblk = Ep // BE
    full = lambda i: (i, 0)
    bcast = lambda i: (0, 0)
    return pl.pallas_call(
        _msg1_body,
        grid=(nblk,),
        in_specs=[
            pl.BlockSpec((BE, HID), full),
            pl.BlockSpec((BE, HID), full),
            pl.BlockSpec((BE, HID), full),
            pl.BlockSpec((1, BE, 1), lambda i: (i, 0, 0)),
            pl.BlockSpec((NTYPE, HID), bcast),
            pl.BlockSpec((NTYPE, HID), bcast),
            pl.BlockSpec((HID, HID), bcast),
            pl.BlockSpec((1, HID), bcast),
            pl.BlockSpec((NTYPE, HID), bcast),
            pl.BlockSpec((HID, HID), bcast),
        ],
        out_specs=pl.BlockSpec((BE, HID), full),
        out_shape=jax.ShapeDtypeStruct((Ep, HID), jnp.float32),
    )(hs0, hs1, hd1, et3, e_tab, rb0, we0, be0, rb1, w1)


def _update_body(h_ref, a0_ref, a1_ref, w_ref, b_ref, out_ref):
    h = h_ref[...]
    pre = _dot(h, w_ref[...]) + a0_ref[...] + a1_ref[...] + b_ref[...]
    out_ref[...] = h + jnp.maximum(pre, 0.0)


def _update(h, a0, a1, w, b):
    nblk = Np // BN
    full = lambda i: (i, 0)
    bcast = lambda i: (0, 0)
    return pl.pallas_call(
        _update_body,
        grid=(nblk,),
        in_specs=[
            pl.BlockSpec((BN, HID), full),
            pl.BlockSpec((BN, HID), full),
            pl.BlockSpec((BN, HID), full),
            pl.BlockSpec((HID, HID), bcast),
            pl.BlockSpec((1, HID), bcast),
        ],
        out_specs=pl.BlockSpec((BN, HID), full),
        out_shape=jax.ShapeDtypeStruct((Np, HID), jnp.float32),
    )(h, a0, a1, w, b)


def kernel(node_feat, edge_feat, edge_index, triplets, h_table, e_table,
           Wmsg, Wself, We, rel_bias, b_h, b_e):
    del triplets
    padE = Ep - E
    zpad = jnp.zeros((padE,), jnp.int32)
    srcp = jnp.concatenate([edge_index[0], zpad])
    dstp = jnp.concatenate([edge_index[1], zpad])
    etp = jnp.concatenate([edge_feat, zpad])
    et3 = etp.reshape(Ep // BE, BE, 1)
    nfp = jnp.concatenate([node_feat, jnp.zeros((Np - N,), jnp.int32)])
    zeros_stripe = jnp.zeros((Np // NS, HID), jnp.float32)
    bh = b_h.reshape(2, 1, HID)
    be = b_e.reshape(2, 1, HID)

    h0 = _sc_gather(h_table, nfp, Np, 64)

    hs0 = _sc_gather(h0, srcp, Ep, C)
    msg0 = _msg0(hs0, et3, e_table, rel_bias[0], Wmsg[0])
    agg0 = _sc_scatter_add(msg0, dstp, zeros_stripe)
    h1 = _update(h0, agg0[0], agg0[1], Wself[0], bh[0])

    both = _sc_gather(h1, jnp.concatenate([srcp, dstp]), 2 * Ep, C)
    hs1 = both[:Ep]
    hd1 = both[Ep:]
    msg1 = _msg1(hs0, hs1, hd1, et3, e_table, rel_bias[0], We[0], be[0],
                 rel_bias[1], Wmsg[1])
    agg1 = _sc_scatter_add(msg1, dstp, zeros_stripe)
    h2 = _update(h1, agg1[0], agg1[1], Wself[1], bh[1])

    return h2[:N]


# speedup vs baseline: 2.6385x; 1.1344x over previous
import functools

import jax
import jax.numpy as jnp
from jax import lax
from jax.experimental import pallas as pl
from jax.experimental.pallas import tpu as pltpu
from jax.experimental.pallas import tpu_sc as plsc

N = 10000
E = 320000
HID = 128
NTYPE = 16

NC = 2
NS = 16
NW = NC * NS
C = 128

Np = 10240
Bn = 12288
Ep = NW * C * 79
BE = 2048
BN = 2048

_MESH = dict(core_axis_name="c", subcore_axis_name="s")


def _sc_gather(table, idx3d, B, K=3):
    nrows = B // C
    nchunks = nrows // NW
    nb = nchunks * C

    @functools.partial(
        pl.kernel,
        mesh=plsc.VectorSubcoreMesh(**_MESH),
        out_type=jax.ShapeDtypeStruct((B, HID), jnp.float32),
        scratch_types=[
            pltpu.VMEM((nchunks, C), jnp.int32),
            pltpu.VMEM((2 * K, C, HID), jnp.float32),
            pltpu.SemaphoreType.DMA,
            pltpu.SemaphoreType.DMA,
        ],
    )
    def gk(table_hbm, idx_hbm, out_hbm, idx_all, rows_v, gsem, wsem):
        wid = lax.axis_index("c") * NS + lax.axis_index("s")
        pltpu.sync_copy(idx_hbm.at[wid], idx_all)
        base_w = wid * nb

        def start_gather(j, b):
            pltpu.async_copy(table_hbm.at[idx_all.at[j]], rows_v.at[b], gsem)

        def drain_gather():
            pltpu.make_async_copy(out_hbm.at[pl.ds(0, C)], rows_v.at[0], gsem).wait()

        def start_write(j, b):
            dst = out_hbm.at[pl.ds(pl.multiple_of(base_w + j * C, 8), C)]
            pltpu.async_copy(rows_v.at[b], dst, wsem)

        def drain_write():
            pltpu.make_async_copy(rows_v.at[0], out_hbm.at[pl.ds(0, C)], wsem).wait()

        def do_group(jbase, s, drain_prev):
            if drain_prev:
                for _ in range(K):
                    drain_write()
            for b in range(K):
                start_gather(jbase + b, s * K + b)
            for _ in range(K):
                drain_gather()
            for b in range(K):
                start_write(jbase + b, s * K + b)

        ngroups = nchunks // K
        tail = nchunks % K
        out_w = 0

        if ngroups >= 2:
            do_group(0, 0, False)
            do_group(K, 1, False)
            out_w += 2 * K
            npairs = (ngroups - 2) // 2
            rem = (ngroups - 2) % 2

            def body(t, carry):
                do_group((2 + 2 * t) * K, 0, True)
                do_group((3 + 2 * t) * K, 1, True)
                return carry

            if npairs:
                lax.fori_loop(0, npairs, body, 0)
            if rem:
                g = ngroups - 1
                do_group(g * K, g % 2, True)
            if tail:
                st = ngroups % 2
                for _ in range(K):
                    drain_write()
                out_w -= K
                for b in range(tail):
                    start_gather(ngroups * K + b, st * K + b)
                for _ in range(tail):
                    drain_gather()
                for b in range(tail):
                    start_write(ngroups * K + b, st * K + b)
                out_w += tail
        else:
            for j in range(nchunks):
                start_gather(j, j)
            for j in range(nchunks):
                drain_gather()
                start_write(j, j)
            out_w = nchunks

        for _ in range(out_w):
            drain_write()

    return gk(table, idx3d)


def _sc_scatter_add(msg, dst3d, zeros_stripe, K=2):
    nchunks = Ep // C // NW
    nb = nchunks * C
    stripe = Np // NS

    @functools.partial(
        pl.kernel,
        mesh=plsc.VectorSubcoreMesh(**_MESH),
        out_type=jax.ShapeDtypeStruct((NC, Np, HID), jnp.float32),
        scratch_types=[
            pltpu.VMEM((nchunks, C), jnp.int32),
            pltpu.VMEM((K, C, HID), jnp.float32),
            pltpu.VMEM_SHARED((Np, HID), jnp.float32),
            pltpu.SemaphoreType.DMA,
        ],
    )
    def sk(msg_hbm, dst_hbm, zeros_hbm, out_hbm, idx_all, rows_v, acc_sh, lsem):
        c = lax.axis_index("c")
        s = lax.axis_index("s")
        wid = c * NS + s

        pltpu.sync_copy(zeros_hbm, acc_sh.at[pl.ds(s * stripe, stripe)])
        pltpu.sync_copy(dst_hbm.at[wid], idx_all)
        plsc.subcore_barrier()

        base_w = wid * nb

        def start_load(j, b):
            src = msg_hbm.at[pl.ds(pl.multiple_of(base_w + j * C, 8), C)]
            pltpu.async_copy(src, rows_v.at[b], lsem)

        def drain_load():
            pltpu.make_async_copy(msg_hbm.at[pl.ds(0, C)], rows_v.at[0], lsem).wait()

        def do_group(jbase, k):
            for b in range(k):
                start_load(jbase + b, b)
            for b in range(k):
                drain_load()
                pltpu.sync_copy(rows_v.at[b], acc_sh.at[idx_all.at[jbase + b]],
                                add=True)

        ngroups = nchunks // K
        tail = nchunks % K

        def body(g, carry):
            do_group(g * K, K)
            return carry

        lax.fori_loop(0, ngroups, body, 0)
        if tail:
            do_group(ngroups * K, tail)

        plsc.subcore_barrier()
        pltpu.sync_copy(acc_sh.at[pl.ds(s * stripe, stripe)],
                        out_hbm.at[c, pl.ds(s * stripe, stripe)])

    return sk(msg, dst3d, zeros_stripe)


def _dot(a, b):
    return jnp.dot(a, b, preferred_element_type=jnp.float32)


def _one_hot(et2):
    t = lax.broadcasted_iota(jnp.int32, (1, NTYPE), 1)
    return (et2 == t).astype(jnp.float32)


def _msg0_body(hs_ref, et_ref, etab_ref, rb_ref, w_ref, out_ref):
    pid = pl.program_id(0)
    oh = _one_hot(et_ref[0])
    e0 = _dot(oh, etab_ref[...])
    rb = _dot(oh, rb_ref[...])
    x = hs_ref[...] * e0 + rb
    msg = _dot(x, w_ref[...])
    rowid = pid * BE + lax.broadcasted_iota(jnp.int32, (BE, 1), 0)
    out_ref[...] = jnp.where(rowid < E, msg, 0.0)


def _msg0(hs0, et3, e_tab, rb0, w0):
    nblk = Ep // BE
    return pl.pallas_call(
        _msg0_body,
        grid=(nblk,),
        in_specs=[
            pl.BlockSpec((BE, HID), lambda i: (i, 0)),
            pl.BlockSpec((1, BE, 1), lambda i: (i, 0, 0)),
            pl.BlockSpec((NTYPE, HID), lambda i: (0, 0)),
            pl.BlockSpec((NTYPE, HID), lambda i: (0, 0)),
            pl.BlockSpec((HID, HID), lambda i: (0, 0)),
        ],
        out_specs=pl.BlockSpec((BE, HID), lambda i: (i, 0)),
        out_shape=jax.ShapeDtypeStruct((Ep, HID), jnp.float32),
    )(hs0, et3, e_tab, rb0, w0)


def _msg1_body(hs0_ref, hs1_ref, hd1_ref, et_ref, etab_ref, rb0_ref, we_ref,
               be_ref, rb1_ref, w1_ref, out_ref):
    pid = pl.program_id(0)
    oh = _one_hot(et_ref[0])
    e0 = _dot(oh, etab_ref[...])
    t = _dot(e0 + hs0_ref[...] + hd1_ref[...], we_ref[...]) + be_ref[...]
    e1 = e0 + jnp.maximum(t, 0.0)
    x = hs1_ref[...] * e1 + _dot(oh, rb1_ref[...])
    msg = _dot(x, w1_ref[...])
    rowid = pid * BE + lax.broadcasted_iota(jnp.int32, (BE, 1), 0)
    out_ref[...] = jnp.where(rowid < E, msg, 0.0)


def _msg1(hs0, hs1, hd1, et3, e_tab, rb0, we0, be0, rb1, w1):
    nblk = Ep // BE
    full = lambda i: (i, 0)
    bcast = lambda i: (0, 0)
    return pl.pallas_call(
        _msg1_body,
        grid=(nblk,),
        in_specs=[
            pl.BlockSpec((BE, HID), full),
            pl.BlockSpec((BE, HID), full),
            pl.BlockSpec((BE, HID), full),
            pl.BlockSpec((1, BE, 1), lambda i: (i, 0, 0)),
            pl.BlockSpec((NTYPE, HID), bcast),
            pl.BlockSpec((NTYPE, HID), bcast),
            pl.BlockSpec((HID, HID), bcast),
            pl.BlockSpec((1, HID), bcast),
            pl.BlockSpec((NTYPE, HID), bcast),
            pl.BlockSpec((HID, HID), bcast),
        ],
        out_specs=pl.BlockSpec((BE, HID), full),
        out_shape=jax.ShapeDtypeStruct((Ep, HID), jnp.float32),
    )(hs0, hs1, hd1, et3, e_tab, rb0, we0, be0, rb1, w1)


def _update_body(h_ref, a0_ref, a1_ref, w_ref, b_ref, out_ref):
    h = h_ref[...]
    pre = _dot(h, w_ref[...]) + a0_ref[...] + a1_ref[...] + b_ref[...]
    out_ref[...] = h + jnp.maximum(pre, 0.0)


def _update(h, a0, a1, w, b):
    nblk = Np // BN
    full = lambda i: (i, 0)
    bcast = lambda i: (0, 0)
    return pl.pallas_call(
        _update_body,
        grid=(nblk,),
        in_specs=[
            pl.BlockSpec((BN, HID), full),
            pl.BlockSpec((BN, HID), full),
            pl.BlockSpec((BN, HID), full),
            pl.BlockSpec((HID, HID), bcast),
            pl.BlockSpec((1, HID), bcast),
        ],
        out_specs=pl.BlockSpec((BN, HID), full),
        out_shape=jax.ShapeDtypeStruct((Np, HID), jnp.float32),
    )(h, a0, a1, w, b)


def kernel(node_feat, edge_feat, edge_index, triplets, h_table, e_table,
           Wmsg, Wself, We, rel_bias, b_h, b_e):
    del triplets
    padE = Ep - E
    zpad = jnp.zeros((padE,), jnp.int32)
    srcp = jnp.concatenate([edge_index[0], zpad])
    dstp = jnp.concatenate([edge_index[1], zpad])
    etp = jnp.concatenate([edge_feat, zpad])
    et3 = etp.reshape(Ep // BE, BE, 1)
    srcp3 = srcp.reshape(NW, Ep // C // NW, C)
    dstp3 = dstp.reshape(NW, Ep // C // NW, C)
    both3 = jnp.concatenate([srcp, dstp]).reshape(NW, 2 * Ep // C // NW, C)
    nfp = jnp.concatenate([node_feat, jnp.zeros((Bn - N,), jnp.int32)])
    zeros_stripe = jnp.zeros((Np // NS, HID), jnp.float32)
    bh = b_h.reshape(2, 1, HID)
    be = b_e.reshape(2, 1, HID)

    h0 = _sc_gather(h_table, nfp.reshape(NW, Bn // C // NW, C), Bn)[:Np]

    hs0 = _sc_gather(h0, srcp3, Ep)
    msg0 = _msg0(hs0, et3, e_table, rel_bias[0], Wmsg[0])
    agg0 = _sc_scatter_add(msg0, dstp3, zeros_stripe)
    h1 = _update(h0, agg0[0], agg0[1], Wself[0], bh[0])

    both = _sc_gather(h1, both3, 2 * Ep)
    hs1 = both[:Ep]
    hd1 = both[Ep:]
    msg1 = _msg1(hs0, hs1, hd1, et3, e_table, rel_bias[0], We[0], be[0],
                 rel_bias[1], Wmsg[1])
    agg1 = _sc_scatter_add(msg1, dstp3, zeros_stripe)
    h2 = _update(h1, agg1[0], agg1[1], Wself[1], bh[1])

    return h2[:N]


# speedup vs baseline: 2.7141x; 1.0287x over previous
import functools

import jax
import jax.numpy as jnp
from jax import lax
from jax.experimental import pallas as pl
from jax.experimental.pallas import tpu as pltpu
from jax.experimental.pallas import tpu_sc as plsc

N = 10000
E = 320000
HID = 128
NTYPE = 16

NC = 2
NS = 16
NW = NC * NS
C = 128

Np = 10240
Bn = 12288
Ep = NW * C * 79
BE = 2048
BN = 2048

_MESH = dict(core_axis_name="c", subcore_axis_name="s")


def _sc_gather(table, idx3d, B, K=3):
    nrows = B // C
    nchunks = nrows // NW
    nb = nchunks * C

    @functools.partial(
        pl.kernel,
        mesh=plsc.VectorSubcoreMesh(**_MESH),
        out_type=jax.ShapeDtypeStruct((B, HID), jnp.float32),
        scratch_types=[
            pltpu.VMEM((nchunks, C), jnp.int32),
            pltpu.VMEM((2 * K, C, HID), jnp.float32),
            pltpu.SemaphoreType.DMA,
            pltpu.SemaphoreType.DMA,
        ],
    )
    def gk(table_hbm, idx_hbm, out_hbm, idx_all, rows_v, gsem, wsem):
        wid = lax.axis_index("c") * NS + lax.axis_index("s")
        pltpu.sync_copy(idx_hbm.at[wid], idx_all)
        base_w = wid * nb

        def start_gather(j, b):
            pltpu.async_copy(table_hbm.at[idx_all.at[j]], rows_v.at[b], gsem)

        def drain_gather():
            pltpu.make_async_copy(out_hbm.at[pl.ds(0, C)], rows_v.at[0], gsem).wait()

        def start_write(j, b):
            dst = out_hbm.at[pl.ds(pl.multiple_of(base_w + j * C, 8), C)]
            pltpu.async_copy(rows_v.at[b], dst, wsem)

        def drain_write():
            pltpu.make_async_copy(rows_v.at[0], out_hbm.at[pl.ds(0, C)], wsem).wait()

        def do_group(jbase, s, drain_prev):
            if drain_prev:
                for _ in range(K):
                    drain_write()
            for b in range(K):
                start_gather(jbase + b, s * K + b)
            for _ in range(K):
                drain_gather()
            for b in range(K):
                start_write(jbase + b, s * K + b)

        ngroups = nchunks // K
        tail = nchunks % K
        out_w = 0

        if ngroups >= 2:
            do_group(0, 0, False)
            do_group(K, 1, False)
            out_w += 2 * K
            npairs = (ngroups - 2) // 2
            rem = (ngroups - 2) % 2

            def body(t, carry):
                do_group((2 + 2 * t) * K, 0, True)
                do_group((3 + 2 * t) * K, 1, True)
                return carry

            if npairs:
                lax.fori_loop(0, npairs, body, 0)
            if rem:
                g = ngroups - 1
                do_group(g * K, g % 2, True)
            if tail:
                st = ngroups % 2
                for _ in range(K):
                    drain_write()
                out_w -= K
                for b in range(tail):
                    start_gather(ngroups * K + b, st * K + b)
                for _ in range(tail):
                    drain_gather()
                for b in range(tail):
                    start_write(ngroups * K + b, st * K + b)
                out_w += tail
        else:
            for j in range(nchunks):
                start_gather(j, j)
            for j in range(nchunks):
                drain_gather()
                start_write(j, j)
            out_w = nchunks

        for _ in range(out_w):
            drain_write()

    return gk(table, idx3d)


def _sc_scatter_add(msg, dst3d, zeros_stripe, CS=128, NSLOT=2):
    nchunks = Ep // CS // NW
    nb = nchunks * CS
    stripe = Np // NS
    ngroups = nchunks // NSLOT
    tail = nchunks % NSLOT
    assert ngroups >= 3

    @functools.partial(
        pl.kernel,
        mesh=plsc.VectorSubcoreMesh(**_MESH),
        out_type=jax.ShapeDtypeStruct((NC, Np, HID), jnp.float32),
        scratch_types=[
            pltpu.VMEM((nchunks, CS), jnp.int32),
            pltpu.VMEM((NSLOT, CS, HID), jnp.float32),
            pltpu.VMEM_SHARED((Np, HID), jnp.float32),
            pltpu.SemaphoreType.DMA,
            pltpu.SemaphoreType.DMA,
        ],
    )
    def sk(msg_hbm, dst_hbm, zeros_hbm, out_hbm, idx_all, rows_v, acc_sh,
           lsem, asem):
        c = lax.axis_index("c")
        s = lax.axis_index("s")
        wid = c * NS + s

        pltpu.sync_copy(zeros_hbm, acc_sh.at[pl.ds(s * stripe, stripe)])
        pltpu.sync_copy(dst_hbm.at[wid], idx_all)
        plsc.subcore_barrier()

        base_w = wid * nb

        def load(j, b):
            src = msg_hbm.at[pl.ds(pl.multiple_of(base_w + j * CS, 8), CS)]
            pltpu.async_copy(src, rows_v.at[b], lsem)

        def drain_l():
            pltpu.make_async_copy(msg_hbm.at[pl.ds(0, CS)], rows_v.at[0],
                                  lsem).wait()

        def add(j, b):
            pltpu.async_copy(rows_v.at[b], acc_sh.at[idx_all.at[j]], asem,
                             add=True)

        def drain_a():
            pltpu.make_async_copy(rows_v.at[0], acc_sh.at[pl.ds(0, CS)],
                                  asem).wait()

        for b in range(NSLOT):
            load(b, b)
        for b in range(NSLOT):
            drain_l()
            add(b, b)
            load(NSLOT + b, b)

        def body(t, carry):
            for b in range(NSLOT):
                j = t * NSLOT + b
                drain_a()
                drain_l()
                add(j, b)
                load(j + NSLOT, b)
            return carry

        lax.fori_loop(1, ngroups - 1, body, 0)

        g = ngroups - 1
        for b in range(NSLOT):
            j = g * NSLOT + b
            drain_a()
            drain_l()
            add(j, b)
            if b < tail:
                load(j + NSLOT, b)
        for b in range(tail):
            j = ngroups * NSLOT + b
            drain_a()
            drain_l()
            add(j, b)
        for _ in range(NSLOT):
            drain_a()

        plsc.subcore_barrier()
        pltpu.sync_copy(acc_sh.at[pl.ds(s * stripe, stripe)],
                        out_hbm.at[c, pl.ds(s * stripe, stripe)])

    return sk(msg, dst3d, zeros_stripe)


def _dot(a, b):
    return jnp.dot(a, b, preferred_element_type=jnp.float32)


def _one_hot(et2):
    t = lax.broadcasted_iota(jnp.int32, (1, NTYPE), 1)
    return (et2 == t).astype(jnp.float32)


def _msg0_body(hs_ref, et_ref, etab_ref, rb_ref, w_ref, out_ref):
    pid = pl.program_id(0)
    oh = _one_hot(et_ref[0])
    e0 = _dot(oh, etab_ref[...])
    rb = _dot(oh, rb_ref[...])
    x = hs_ref[...] * e0 + rb
    msg = _dot(x, w_ref[...])
    rowid = pid * BE + lax.broadcasted_iota(jnp.int32, (BE, 1), 0)
    out_ref[...] = jnp.where(rowid < E, msg, 0.0)


def _msg0(hs0, et3, e_tab, rb0, w0):
    nblk = Ep // BE
    return pl.pallas_call(
        _msg0_body,
        grid=(nblk,),
        in_specs=[
            pl.BlockSpec((BE, HID), lambda i: (i, 0)),
            pl.BlockSpec((1, BE, 1), lambda i: (i, 0, 0)),
            pl.BlockSpec((NTYPE, HID), lambda i: (0, 0)),
            pl.BlockSpec((NTYPE, HID), lambda i: (0, 0)),
            pl.BlockSpec((HID, HID), lambda i: (0, 0)),
        ],
        out_specs=pl.BlockSpec((BE, HID), lambda i: (i, 0)),
        out_shape=jax.ShapeDtypeStruct((Ep, HID), jnp.float32),
    )(hs0, et3, e_tab, rb0, w0)


def _msg1_body(hs0_ref, hs1_ref, hd1_ref, et_ref, etab_ref, rb0_ref, we_ref,
               be_ref, rb1_ref, w1_ref, out_ref):
    pid = pl.program_id(0)
    oh = _one_hot(et_ref[0])
    e0 = _dot(oh, etab_ref[...])
    t = _dot(e0 + hs0_ref[...] + hd1_ref[...], we_ref[...]) + be_ref[...]
    e1 = e0 + jnp.maximum(t, 0.0)
    x = hs1_ref[...] * e1 + _dot(oh, rb1_ref[...])
    msg = _dot(x, w1_ref[...])
    rowid = pid * BE + lax.broadcasted_iota(jnp.int32, (BE, 1), 0)
    out_ref[...] = jnp.where(rowid < E, msg, 0.0)


def _msg1(hs0, hs1, hd1, et3, e_tab, rb0, we0, be0, rb1, w1):
    nblk = Ep // BE
    full = lambda i: (i, 0)
    bcast = lambda i: (0, 0)
    return pl.pallas_call(
        _msg1_body,
        grid=(nblk,),
        in_specs=[
            pl.BlockSpec((BE, HID), full),
            pl.BlockSpec((BE, HID), full),
            pl.BlockSpec((BE, HID), full),
            pl.BlockSpec((1, BE, 1), lambda i: (i, 0, 0)),
            pl.BlockSpec((NTYPE, HID), bcast),
            pl.BlockSpec((NTYPE, HID), bcast),
            pl.BlockSpec((HID, HID), bcast),
            pl.BlockSpec((1, HID), bcast),
            pl.BlockSpec((NTYPE, HID), bcast),
            pl.BlockSpec((HID, HID), bcast),
        ],
        out_specs=pl.BlockSpec((BE, HID), full),
        out_shape=jax.ShapeDtypeStruct((Ep, HID), jnp.float32),
    )(hs0, hs1, hd1, et3, e_tab, rb0, we0, be0, rb1, w1)


def _update_body(h_ref, a0_ref, a1_ref, w_ref, b_ref, out_ref):
    h = h_ref[...]
    pre = _dot(h, w_ref[...]) + a0_ref[...] + a1_ref[...] + b_ref[...]
    out_ref[...] = h + jnp.maximum(pre, 0.0)


def _update(h, a0, a1, w, b):
    nblk = Np // BN
    full = lambda i: (i, 0)
    bcast = lambda i: (0, 0)
    return pl.pallas_call(
        _update_body,
        grid=(nblk,),
        in_specs=[
            pl.BlockSpec((BN, HID), full),
            pl.BlockSpec((BN, HID), full),
            pl.BlockSpec((BN, HID), full),
            pl.BlockSpec((HID, HID), bcast),
            pl.BlockSpec((1, HID), bcast),
        ],
        out_specs=pl.BlockSpec((BN, HID), full),
        out_shape=jax.ShapeDtypeStruct((Np, HID), jnp.float32),
    )(h, a0, a1, w, b)


def kernel(node_feat, edge_feat, edge_index, triplets, h_table, e_table,
           Wmsg, Wself, We, rel_bias, b_h, b_e):
    del triplets
    padE = Ep - E
    zpad = jnp.zeros((padE,), jnp.int32)
    srcp = jnp.concatenate([edge_index[0], zpad])
    dstp = jnp.concatenate([edge_index[1], zpad])
    etp = jnp.concatenate([edge_feat, zpad])
    et3 = etp.reshape(Ep // BE, BE, 1)
    srcp3 = srcp.reshape(NW, Ep // C // NW, C)
    dstp3 = dstp.reshape(NW, Ep // C // NW, C)
    both3 = jnp.concatenate([srcp, dstp]).reshape(NW, 2 * Ep // C // NW, C)
    nfp = jnp.concatenate([node_feat, jnp.zeros((Bn - N,), jnp.int32)])
    zeros_stripe = jnp.zeros((Np // NS, HID), jnp.float32)
    bh = b_h.reshape(2, 1, HID)
    be = b_e.reshape(2, 1, HID)

    h0 = _sc_gather(h_table, nfp.reshape(NW, Bn // C // NW, C), Bn)[:Np]

    hs0 = _sc_gather(h0, srcp3, Ep)
    msg0 = _msg0(hs0, et3, e_table, rel_bias[0], Wmsg[0])
    agg0 = _sc_scatter_add(msg0, dstp3, zeros_stripe)
    h1 = _update(h0, agg0[0], agg0[1], Wself[0], bh[0])

    both = _sc_gather(h1, both3, 2 * Ep)
    hs1 = both[:Ep]
    hd1 = both[Ep:]
    msg1 = _msg1(hs0, hs1, hd1, et3, e_table, rel_bias[0], We[0], be[0],
                 rel_bias[1], Wmsg[1])
    agg1 = _sc_scatter_add(msg1, dstp3, zeros_stripe)
    h2 = _update(h1, agg1[0], agg1[1], Wself[1], bh[1])

    return h2[:N]


# speedup vs baseline: 3.2904x; 1.2124x over previous
import functools

import jax
import jax.numpy as jnp
from jax import lax
from jax.experimental import pallas as pl
from jax.experimental.pallas import tpu as pltpu
from jax.experimental.pallas import tpu_sc as plsc

N = 10000
E = 320000
HID = 128
NTYPE = 16

NC = 2
NS = 16
NW = NC * NS
C = 128

Np = 10240
Bn = 12288
Ep = NW * C * 79
BE = 2048
BN = 2048

_MESH = dict(core_axis_name="c", subcore_axis_name="s")


def _sc_gather(table_a, table_b, idx3d, B, K=3):
    nrows = B // C
    nchunks = nrows // NW
    nb = nchunks * C

    @functools.partial(
        pl.kernel,
        mesh=plsc.VectorSubcoreMesh(**_MESH),
        out_type=jax.ShapeDtypeStruct((B, HID), jnp.float32),
        scratch_types=[
            pltpu.VMEM((nchunks, C), jnp.int32),
            pltpu.VMEM((2 * K, C, HID), jnp.float32),
            pltpu.SemaphoreType.DMA,
            pltpu.SemaphoreType.DMA,
        ],
    )
    def gk(taba_hbm, tabb_hbm, idx_hbm, out_hbm, idx_all, rows_v, gsem, wsem):
        c = lax.axis_index("c")
        wid = c * NS + lax.axis_index("s")
        pltpu.sync_copy(idx_hbm.at[wid], idx_all)
        base_w = wid * nb

        def run(table_hbm):
            def start_gather(j, b):
                pltpu.async_copy(table_hbm.at[idx_all.at[j]], rows_v.at[b],
                                 gsem)

            def drain_gather():
                pltpu.make_async_copy(out_hbm.at[pl.ds(0, C)], rows_v.at[0],
                                      gsem).wait()

            def start_write(j, b):
                dst = out_hbm.at[pl.ds(pl.multiple_of(base_w + j * C, 8), C)]
                pltpu.async_copy(rows_v.at[b], dst, wsem)

            def drain_write():
                pltpu.make_async_copy(rows_v.at[0], out_hbm.at[pl.ds(0, C)],
                                      wsem).wait()

            def do_group(jbase, s, drain_prev):
                if drain_prev:
                    for _ in range(K):
                        drain_write()
                for b in range(K):
                    start_gather(jbase + b, s * K + b)
                for _ in range(K):
                    drain_gather()
                for b in range(K):
                    start_write(jbase + b, s * K + b)

            ngroups = nchunks // K
            tail = nchunks % K
            out_w = 0

            if ngroups >= 2:
                do_group(0, 0, False)
                do_group(K, 1, False)
                out_w += 2 * K
                npairs = (ngroups - 2) // 2
                rem = (ngroups - 2) % 2

                def body(t, carry):
                    do_group((2 + 2 * t) * K, 0, True)
                    do_group((3 + 2 * t) * K, 1, True)
                    return carry

                if npairs:
                    lax.fori_loop(0, npairs, body, 0)
                if rem:
                    g = ngroups - 1
                    do_group(g * K, g % 2, True)
                if tail:
                    st = ngroups % 2
                    for _ in range(K):
                        drain_write()
                    out_w -= K
                    for b in range(tail):
                        start_gather(ngroups * K + b, st * K + b)
                    for _ in range(tail):
                        drain_gather()
                    for b in range(tail):
                        start_write(ngroups * K + b, st * K + b)
                    out_w += tail
            else:
                for j in range(nchunks):
                    start_gather(j, j)
                for j in range(nchunks):
                    drain_gather()
                    start_write(j, j)
                out_w = nchunks

            for _ in range(out_w):
                drain_write()

        @pl.when(c == 0)
        def _():
            run(taba_hbm)

        @pl.when(c == 1)
        def _():
            run(tabb_hbm)

    return gk(table_a, table_b, idx3d)


def _sc_scatter_add(msg, dst3d, zeros_stripe, CS=128, NSLOT=2):
    nchunks = Ep // CS // NW
    nb = nchunks * CS
    stripe = Np // NS
    ngroups = nchunks // NSLOT
    tail = nchunks % NSLOT
    assert ngroups >= 3

    @functools.partial(
        pl.kernel,
        mesh=plsc.VectorSubcoreMesh(**_MESH),
        out_type=jax.ShapeDtypeStruct((NC, Np, HID), jnp.float32),
        scratch_types=[
            pltpu.VMEM((nchunks, CS), jnp.int32),
            pltpu.VMEM((NSLOT, CS, HID), jnp.float32),
            pltpu.VMEM_SHARED((Np, HID), jnp.float32),
            pltpu.SemaphoreType.DMA,
            pltpu.SemaphoreType.DMA,
        ],
    )
    def sk(msg_hbm, dst_hbm, zeros_hbm, out_hbm, idx_all, rows_v, acc_sh,
           lsem, asem):
        c = lax.axis_index("c")
        s = lax.axis_index("s")
        wid = c * NS + s

        pltpu.sync_copy(zeros_hbm, acc_sh.at[pl.ds(s * stripe, stripe)])
        pltpu.sync_copy(dst_hbm.at[wid], idx_all)
        plsc.subcore_barrier()

        base_w = wid * nb

        def load(j, b):
            src = msg_hbm.at[pl.ds(pl.multiple_of(base_w + j * CS, 8), CS)]
            pltpu.async_copy(src, rows_v.at[b], lsem)

        def drain_l():
            pltpu.make_async_copy(msg_hbm.at[pl.ds(0, CS)], rows_v.at[0],
                                  lsem).wait()

        def add(j, b):
            pltpu.async_copy(rows_v.at[b], acc_sh.at[idx_all.at[j]], asem,
                             add=True)

        def drain_a():
            pltpu.make_async_copy(rows_v.at[0], acc_sh.at[pl.ds(0, CS)],
                                  asem).wait()

        for b in range(NSLOT):
            load(b, b)
        for b in range(NSLOT):
            drain_l()
            add(b, b)
            load(NSLOT + b, b)

        def body(t, carry):
            for b in range(NSLOT):
                j = t * NSLOT + b
                drain_a()
                drain_l()
                add(j, b)
                load(j + NSLOT, b)
            return carry

        lax.fori_loop(1, ngroups - 1, body, 0)

        g = ngroups - 1
        for b in range(NSLOT):
            j = g * NSLOT + b
            drain_a()
            drain_l()
            add(j, b)
            if b < tail:
                load(j + NSLOT, b)
        for b in range(tail):
            j = ngroups * NSLOT + b
            drain_a()
            drain_l()
            add(j, b)
        for _ in range(NSLOT):
            drain_a()

        plsc.subcore_barrier()
        pltpu.sync_copy(acc_sh.at[pl.ds(s * stripe, stripe)],
                        out_hbm.at[c, pl.ds(s * stripe, stripe)])

    return sk(msg, dst3d, zeros_stripe)


def _dot(a, b):
    return jnp.dot(a, b, preferred_element_type=jnp.float32)


def _one_hot(et2):
    t = lax.broadcasted_iota(jnp.int32, (1, NTYPE), 1)
    return (et2 == t).astype(jnp.float32)


def _msg0_body(hs_ref, et_ref, etab_ref, rb_ref, w_ref, out_ref):
    pid = pl.program_id(0)
    oh = _one_hot(et_ref[0])
    e0 = _dot(oh, etab_ref[...])
    rb = _dot(oh, rb_ref[...])
    x = hs_ref[...] * e0 + rb
    msg = _dot(x, w_ref[...])
    rowid = pid * BE + lax.broadcasted_iota(jnp.int32, (BE, 1), 0)
    out_ref[...] = jnp.where(rowid < E, msg, 0.0)


def _msg0(hs0, et3, e_tab, rb0, w0):
    nblk = Ep // BE
    return pl.pallas_call(
        _msg0_body,
        grid=(nblk,),
        in_specs=[
            pl.BlockSpec((BE, HID), lambda i: (i, 0)),
            pl.BlockSpec((1, BE, 1), lambda i: (i, 0, 0)),
            pl.BlockSpec((NTYPE, HID), lambda i: (0, 0)),
            pl.BlockSpec((NTYPE, HID), lambda i: (0, 0)),
            pl.BlockSpec((HID, HID), lambda i: (0, 0)),
        ],
        out_specs=pl.BlockSpec((BE, HID), lambda i: (i, 0)),
        out_shape=jax.ShapeDtypeStruct((Ep, HID), jnp.float32),
    )(hs0, et3, e_tab, rb0, w0)


def _msg1_body(hs0_ref, hs1_ref, hd1_ref, et_ref, etab_ref, rb0_ref, we_ref,
               be_ref, rb1_ref, w1_ref, out_ref):
    pid = pl.program_id(0)
    oh = _one_hot(et_ref[0])
    e0 = _dot(oh, etab_ref[...])
    t = _dot(e0 + hs0_ref[...] + hd1_ref[...], we_ref[...]) + be_ref[...]
    e1 = e0 + jnp.maximum(t, 0.0)
    x = hs1_ref[...] * e1 + _dot(oh, rb1_ref[...])
    msg = _dot(x, w1_ref[...])
    rowid = pid * BE + lax.broadcasted_iota(jnp.int32, (BE, 1), 0)
    out_ref[...] = jnp.where(rowid < E, msg, 0.0)


def _msg1(hs0, both, et3, e_tab, rb0, we0, be0, rb1, w1):
    nblk = Ep // BE
    full = lambda i: (i, 0)
    bcast = lambda i: (0, 0)
    return pl.pallas_call(
        _msg1_body,
        grid=(nblk,),
        in_specs=[
            pl.BlockSpec((BE, HID), full),
            pl.BlockSpec((BE, HID), full),
            pl.BlockSpec((BE, HID), lambda i: (i + Ep // BE, 0)),
            pl.BlockSpec((1, BE, 1), lambda i: (i, 0, 0)),
            pl.BlockSpec((NTYPE, HID), bcast),
            pl.BlockSpec((NTYPE, HID), bcast),
            pl.BlockSpec((HID, HID), bcast),
            pl.BlockSpec((1, HID), bcast),
            pl.BlockSpec((NTYPE, HID), bcast),
            pl.BlockSpec((HID, HID), bcast),
        ],
        out_specs=pl.BlockSpec((BE, HID), full),
        out_shape=jax.ShapeDtypeStruct((Ep, HID), jnp.float32),
    )(hs0, both, both, et3, e_tab, rb0, we0, be0, rb1, w1)


def _update_body(h_ref, a0_ref, a1_ref, w_ref, b_ref, out_ref, copy_ref):
    h = h_ref[...]
    pre = _dot(h, w_ref[...]) + a0_ref[...] + a1_ref[...] + b_ref[...]
    hn = h + jnp.maximum(pre, 0.0)
    out_ref[...] = hn
    if copy_ref is not None:
        copy_ref[...] = hn


def _update(h, a0, a1, w, b, want_copy=False):
    nblk = Np // BN
    full = lambda i: (i, 0)
    bcast = lambda i: (0, 0)
    n_out = 2 if want_copy else 1
    body = (_update_body if want_copy
            else lambda *a: _update_body(*a, None))
    out = pl.pallas_call(
        body,
        grid=(nblk,),
        in_specs=[
            pl.BlockSpec((BN, HID), full),
            pl.BlockSpec((BN, HID), full),
            pl.BlockSpec((BN, HID), full),
            pl.BlockSpec((HID, HID), bcast),
            pl.BlockSpec((1, HID), bcast),
        ],
        out_specs=[pl.BlockSpec((BN, HID), full)] * n_out,
        out_shape=[jax.ShapeDtypeStruct((Np, HID), jnp.float32)] * n_out,
    )(h, a0, a1, w, b)
    return out if want_copy else out[0]


def _dup_body(x_ref, o1_ref, o2_ref):
    x = x_ref[...]
    o1_ref[...] = x
    o2_ref[...] = x


def _dup(x):
    n, d = x.shape
    nblk = n // BN if n % BN == 0 else 1
    blk = n // nblk
    full = lambda i: (i, 0)
    return pl.pallas_call(
        _dup_body,
        grid=(nblk,),
        in_specs=[pl.BlockSpec((blk, d), full)],
        out_specs=[pl.BlockSpec((blk, d), full)] * 2,
        out_shape=[jax.ShapeDtypeStruct((n, d), x.dtype)] * 2,
    )(x)


def kernel(node_feat, edge_feat, edge_index, triplets, h_table, e_table,
           Wmsg, Wself, We, rel_bias, b_h, b_e):
    del triplets
    padE = Ep - E
    zpad = jnp.zeros((padE,), jnp.int32)
    srcp = jnp.concatenate([edge_index[0], zpad])
    dstp = jnp.concatenate([edge_index[1], zpad])
    etp = jnp.concatenate([edge_feat, zpad])
    et3 = etp.reshape(Ep // BE, BE, 1)
    srcp3 = srcp.reshape(NW, Ep // C // NW, C)
    dstp3 = dstp.reshape(NW, Ep // C // NW, C)
    both3 = jnp.concatenate([srcp, dstp]).reshape(NW, 2 * Ep // C // NW, C)
    nfp = jnp.concatenate([node_feat, jnp.zeros((Bn - N,), jnp.int32)])
    zeros_stripe = jnp.zeros((Np // NS, HID), jnp.float32)
    bh = b_h.reshape(2, 1, HID)
    be = b_e.reshape(2, 1, HID)

    hta, htb = _dup(h_table)
    h0 = _sc_gather(hta, htb, nfp.reshape(NW, Bn // C // NW, C), Bn)[:Np]

    h0a, h0b = _dup(h0)
    hs0 = _sc_gather(h0a, h0b, srcp3, Ep)
    msg0 = _msg0(hs0, et3, e_table, rel_bias[0], Wmsg[0])
    agg0 = _sc_scatter_add(msg0, dstp3, zeros_stripe)
    h1, h1b = _update(h0, agg0[0], agg0[1], Wself[0], bh[0], want_copy=True)

    both = _sc_gather(h1, h1b, both3, 2 * Ep)
    msg1 = _msg1(hs0, both, et3, e_table, rel_bias[0], We[0], be[0],
                 rel_bias[1], Wmsg[1])
    agg1 = _sc_scatter_add(msg1, dstp3, zeros_stripe)
    h2 = _update(h1, agg1[0], agg1[1], Wself[1], bh[1])

    return h2[:N]
